# Initial kernel scaffold; baseline (speedup 1.0000x reference)
import jax
import jax.numpy as jnp
from jax.experimental import pallas as pl


def kernel(x, adj1_indices, adj1_values, adj2_indices, adj2_values, W1):
    raise NotImplementedError("write your pallas kernel here")


# speedup vs baseline: 3.7138x; 3.7138x over previous
import functools

import jax
import jax.numpy as jnp
from jax import lax
from jax.experimental import pallas as pl
from jax.experimental.pallas import tpu as pltpu
from jax.experimental.pallas import tpu_sc as plsc

N = 10000
D = 128
E = 320000
NS = 16
NC = 2
N_PAD = 10240
RPT = N_PAD // NS
ZCH = 80
EPT = E // NS
CH = 80
NCHUNK = EPT // CH


def _spmm_body(x_hbm, rows_hbm, cols_hbm, vals_hbm, out_hbm,
               acc, cols_v, ridx_v, vals_v, gbuf, zbuf, sem):
    c = lax.axis_index("c")
    s = lax.axis_index("s")
    zero = jnp.zeros((16,), jnp.float32)

    def _zrow(r, _):
        for f in range(D // 16):
            zbuf[r, pl.ds(f * 16, 16)] = zero
        return 0
    lax.fori_loop(0, ZCH, _zrow, 0)
    row0 = s * RPT
    for z in range(RPT // ZCH):
        pltpu.sync_copy(zbuf, acc.at[pl.ds(row0 + z * ZCH, ZCH)])
    plsc.subcore_barrier()

    ebase = c * E + s * EPT

    def _chunk(i, _):
        off = ebase + i * CH
        pltpu.sync_copy(cols_hbm.at[pl.ds(off, CH)], cols_v)
        pltpu.sync_copy(rows_hbm.at[pl.ds(off, CH)], ridx_v)
        pltpu.sync_copy(vals_hbm.at[pl.ds(off, CH)], vals_v)
        pltpu.async_copy(x_hbm.at[cols_v], gbuf, sem).wait()

        def _scale(g, _):
            vv = vals_v[pl.ds(g * 16, 16)]
            for j in range(16):
                vsplat = jnp.broadcast_to(vv[j], (16,))
                e = g * 16 + j
                for f in range(D // 16):
                    gbuf[e, pl.ds(f * 16, 16)] = (
                        gbuf[e, pl.ds(f * 16, 16)] * vsplat)
            return 0
        lax.fori_loop(0, CH // 16, _scale, 0)

        pltpu.sync_copy(gbuf, acc.at[ridx_v], add=True)
        return 0
    lax.fori_loop(0, NCHUNK, _chunk, 0)
    plsc.subcore_barrier()

    for z in range(RPT // ZCH):
        r = row0 + z * ZCH
        pltpu.sync_copy(acc.at[pl.ds(r, ZCH)], out_hbm.at[c, pl.ds(r, ZCH)])


@functools.cache
def _make_spmm():
    return pl.kernel(
        _spmm_body,
        out_type=jax.ShapeDtypeStruct((NC, N_PAD, D), jnp.float32),
        mesh=plsc.VectorSubcoreMesh(
            core_axis_name="c", subcore_axis_name="s",
            num_cores=NC, num_subcores=NS),
        scratch_types=[
            pltpu.VMEM_SHARED((N_PAD, D), jnp.float32),
            pltpu.VMEM((CH,), jnp.int32),
            pltpu.VMEM((CH,), jnp.int32),
            pltpu.VMEM((CH,), jnp.float32),
            pltpu.VMEM((CH, D), jnp.float32),
            pltpu.VMEM((ZCH, D), jnp.float32),
            pltpu.SemaphoreType.DMA,
        ],
    )


BLK = 400


def _proj_body(x_ref, s_ref, w_ref, o_ref):
    w = w_ref[...]
    dn = (((1,), (1,)), ((), ()))
    o_ref[:, 0:D] = lax.dot_general(
        x_ref[...], w, dn, preferred_element_type=jnp.float32)
    o_ref[:, D:2 * D] = lax.dot_general(
        s_ref[0], w, dn, preferred_element_type=jnp.float32)
    o_ref[:, 2 * D:3 * D] = lax.dot_general(
        s_ref[1], w, dn, preferred_element_type=jnp.float32)


def _proj(x, s, w):
    return pl.pallas_call(
        _proj_body,
        grid=(N // BLK,),
        in_specs=[
            pl.BlockSpec((BLK, D), lambda i: (i, 0)),
            pl.BlockSpec((NC, BLK, D), lambda i: (0, i, 0)),
            pl.BlockSpec((D, D), lambda i: (0, 0)),
        ],
        out_specs=pl.BlockSpec((BLK, 3 * D), lambda i: (i, 0)),
        out_shape=jax.ShapeDtypeStruct((N, 3 * D), jnp.float32),
    )(x, s, w)


def kernel(x, adj1_indices, adj1_values, adj2_indices, adj2_values, W1):
    rows = jnp.concatenate([adj1_indices[0], adj2_indices[0]]).astype(jnp.int32)
    cols = jnp.concatenate([adj1_indices[1], adj2_indices[1]]).astype(jnp.int32)
    vals = jnp.concatenate([adj1_values, adj2_values])
    s = _make_spmm()(x, rows, cols, vals)
    return _proj(x, s, W1)
